# baseline (speedup 1.0000x reference)
import functools

import jax
import jax.numpy as jnp
from jax import lax
from jax.experimental import pallas as pl
from jax.experimental.pallas import tpu as pltpu
from jax.experimental.pallas import tpu_sc as plsc

B = 16384
D = 64
NC = 2
NS = 16
NW = NC * NS
V = 1000000
VPAD = 1000064
WLANES = 256
TW = (V - 1) // WLANES
CS_MAX = VPAD - WLANES
P = TW // NW + 1
ORING = 16
SENT = 127 << 22

_sc_mesh = plsc.VectorSubcoreMesh(core_axis_name="c", subcore_axis_name="s")


@functools.partial(
    pl.kernel,
    out_type=[
        jax.ShapeDtypeStruct((B + 8, 128), jnp.float32),
        jax.ShapeDtypeStruct((B + 8, 128), jnp.float32),
    ],
    mesh=_sc_mesh,
    compiler_params=pltpu.CompilerParams(
        use_tc_tiling_on_sc=True, needs_layout_passes=False),
    scratch_types=[
        pltpu.VMEM((B,), jnp.int32),
        pltpu.VMEM((B + 16,), jnp.int32),
        pltpu.VMEM((B + 32,), jnp.int32),
        pltpu.VMEM((144,), jnp.int32),
        pltpu.VMEM((144,), jnp.int32),
        pltpu.VMEM((144,), jnp.int32),
        pltpu.VMEM((2, D, WLANES), jnp.float32),
        pltpu.VMEM((2, 16, 128), jnp.float32),
        pltpu.SemaphoreType.DMA,
        pltpu.SemaphoreType.DMA,
    ],
)
def _sc_gather(uidx, bidx, utabT, btabT, uout, bout,
               stage, lst, slst, hist, ioff, wkoff, wbuf, orow, wsem, osem):
    wid = lax.axis_index("s") * NC + lax.axis_index("c")
    iota16 = lax.iota(jnp.int32, 16)
    lane0 = iota16 == 0

    def scalar(vec, k):
        return lax.squeeze(lax.slice(vec, (k,), (k + 1,)), (0,))

    def splat_at(ref, i):
        return scalar(plsc.load_gather(ref, [jnp.full((16,), i, jnp.int32)]), 0)

    def route(idx_hbm):
        pltpu.sync_copy(idx_hbm, stage)

        def rbody(j4, cnt):
            parts = []
            for u in range(4):
                j = j4 * 4 + u
                iv = stage[pl.ds(j * 16, 16)]
                t = lax.shift_right_logical(iv, 8)
                mask = lax.bitwise_and(t, 31) == wid
                packed = lax.bitwise_or(
                    lax.shift_left(lax.shift_right_logical(t, 5), 22),
                    lax.bitwise_or(
                        lax.shift_left(lax.bitwise_and(iv, 255), 14),
                        iota16 + j * 16))
                nm = plsc.all_reduce_population_count(mask)
                parts.append((packed, mask, nm))
            for packed, mask, nm in parts:
                plsc.store_compressed(lst.at[pl.ds(cnt, 16)], packed, mask=mask)
                cnt = cnt + scalar(nm, 0)
            return cnt

        cnt = lax.fori_loop(0, B // 64, rbody, jnp.int32(0))

        for j in range(9):
            hist[pl.ds(j * 16, 16)] = jnp.zeros((16,), jnp.int32)

        nch = lax.div(cnt + jnp.int32(15), jnp.int32(16))

        def hbody(j, carry):
            pv = lst[pl.ds(j * 16, 16)]
            vi = (iota16 + j * 16) < cnt
            p_vec = jnp.where(vi, lax.shift_right_logical(pv, 22), 0)
            rank, lastm = plsc.scan_count(p_vec, mask=vi)
            cur = plsc.load_gather(hist, [p_vec])
            plsc.store_scatter(hist, [p_vec], cur + rank,
                               mask=jnp.logical_and(lastm, vi))
            return carry

        lax.fori_loop(0, nch, hbody, 0)

        def csum(j, carry):
            hv = hist[pl.ds(j * 16, 16)]
            cs = plsc.cumsum(hv) + carry
            ioff[pl.ds(j * 16, 16)] = cs
            wkoff[pl.ds(j * 16, 16)] = cs - hv
            return scalar(cs, 15)

        lax.fori_loop(0, 9, csum, jnp.int32(0))

        def sbody(j, carry):
            pv = lst[pl.ds(j * 16, 16)]
            vi = (iota16 + j * 16) < cnt
            p_vec = jnp.where(vi, lax.shift_right_logical(pv, 22), 0)
            rank, lastm = plsc.scan_count(p_vec, mask=vi)
            base = plsc.load_gather(wkoff, [p_vec])
            plsc.store_scatter(slst, [jnp.where(vi, base + rank - 1, 0)], pv,
                               mask=vi)
            plsc.store_scatter(wkoff, [p_vec], base + rank,
                               mask=jnp.logical_and(lastm, vi))
            return carry

        lax.fori_loop(0, nch, sbody, 0)
        slst[pl.ds(cnt, 16)] = jnp.full((16,), SENT, jnp.int32)
        return cnt

    def run(tabT, out_hbm, ocnt0):
        def fetch(p, slot):
            t = jnp.minimum(p * NW + wid, TW)
            cs = pl.multiple_of(jnp.minimum(t * WLANES, CS_MAX), 128)
            pltpu.async_copy(tabT.at[:, pl.ds(cs, WLANES)], wbuf.at[slot], wsem)

        fetch(0, 0)

        def pbody(p, ocnt):
            slot = lax.rem(p, 2)

            @pl.when(p + 1 < P)
            def _():
                fetch(p + 1, 1 - slot)

            pltpu.make_async_copy(
                tabT.at[:, pl.ds(0, WLANES)], wbuf.at[slot], wsem).wait()
            t = jnp.minimum(p * NW + wid, TW)
            cs = jnp.minimum(t * WLANES, CS_MAX)
            tcor = t * WLANES - cs
            seg_s = jnp.where(p > 0, splat_at(ioff, jnp.maximum(p - 1, 0)),
                              jnp.int32(0))
            seg_e = splat_at(ioff, p)

            def cbody(j, ocnt):
                pv = slst[pl.ds(j * 16, 16)]
                pmatch = lax.shift_right_logical(pv, 22) == p
                off_vec = lax.bitwise_and(
                    lax.shift_right_logical(pv, 14), 255) + tcor
                off_vec = jnp.where(pmatch, off_vec, 0)
                pos_vec = jnp.where(pmatch,
                                    lax.bitwise_and(pv, jnp.int32(16383)),
                                    jnp.int32(B))
                oslot = lax.rem(ocnt, 2)

                @pl.when(ocnt >= 2)
                def _():
                    pltpu.make_async_copy(
                        out_hbm.at[pl.ds(0, 16)], orow.at[oslot], osem).wait()

                for r in range(D):
                    vals = plsc.load_gather(
                        wbuf.at[slot],
                        [jnp.full((16,), r, jnp.int32), off_vec])
                    plsc.store_scatter(
                        orow.at[oslot],
                        [iota16, jnp.full((16,), r, jnp.int32)], vals)
                pltpu.async_copy(orow.at[oslot], out_hbm.at[pos_vec], osem)
                return ocnt + 1

            return lax.fori_loop(
                lax.shift_right_logical(seg_s, 4),
                lax.shift_right_logical(seg_e + jnp.int32(15), 4),
                cbody, ocnt)

        return lax.fori_loop(0, P, pbody, ocnt0)

    route(uidx)
    ocnt = run(utabT, uout, jnp.int32(0))
    route(bidx)
    ocnt = run(btabT, bout, ocnt)

    def dbody(m, carry):
        pltpu.make_async_copy(
            uout.at[pl.ds(0, 16)], orow.at[0], osem).wait()
        return carry

    lax.fori_loop(0, jnp.minimum(ocnt, 2), dbody, 0)


BLK = 2048


def _mlp_body(u_ref, b_ref, w1u_ref, w1b_ref, b1_ref, w2_ref, b2_ref,
              w3_ref, b3_ref, w4_ref, b4_ref, out_ref):
    u = u_ref[:, :D]
    b = b_ref[:, :D]
    h = (jnp.dot(u, w1u_ref[...], preferred_element_type=jnp.float32)
         + jnp.dot(b, w1b_ref[...], preferred_element_type=jnp.float32)
         + b1_ref[...])
    h = jnp.maximum(h, 0.0)
    h = jnp.dot(h, w2_ref[...], preferred_element_type=jnp.float32) + b2_ref[...]
    h = jnp.maximum(h, 0.0)
    h = jnp.dot(h, w3_ref[...], preferred_element_type=jnp.float32) + b3_ref[...]
    h = jnp.maximum(h, 0.0)
    o = jnp.dot(h, w4_ref[...], preferred_element_type=jnp.float32) + b4_ref[...]
    out_ref[...] = jax.nn.sigmoid(o)


_mlp_call = pl.pallas_call(
    _mlp_body,
    grid=(B // BLK,),
    in_specs=[
        pl.BlockSpec((BLK, 128), lambda i: (i, 0)),
        pl.BlockSpec((BLK, 128), lambda i: (i, 0)),
        pl.BlockSpec((D, 128), lambda i: (0, 0)),
        pl.BlockSpec((D, 128), lambda i: (0, 0)),
        pl.BlockSpec((1, 128), lambda i: (0, 0)),
        pl.BlockSpec((128, 64), lambda i: (0, 0)),
        pl.BlockSpec((1, 64), lambda i: (0, 0)),
        pl.BlockSpec((64, 32), lambda i: (0, 0)),
        pl.BlockSpec((1, 32), lambda i: (0, 0)),
        pl.BlockSpec((32, 1), lambda i: (0, 0)),
        pl.BlockSpec((1, 1), lambda i: (0, 0)),
    ],
    out_specs=pl.BlockSpec((BLK, 1), lambda i: (i, 0)),
    out_shape=jax.ShapeDtypeStruct((B, 1), jnp.float32),
)


def kernel(user, business, user_table, business_table, W1, b1, W2, b2, W3, b3, W4, b4):
    uidx = user.astype(jnp.int32)
    bidx = business.astype(jnp.int32)
    u_emb, b_emb = _sc_gather(uidx, bidx, user_table.T, business_table.T)
    w1ut = W1[:, :D].T
    w1bt = W1[:, D:].T
    out = _mlp_call(u_emb, b_emb, w1ut, w1bt, b1.reshape(1, 128),
                    W2.T, b2.reshape(1, 64), W3.T, b3.reshape(1, 32),
                    W4.T, b4.reshape(1, 1))
    return out[:, 0]


# speedup vs baseline: 1.0062x; 1.0062x over previous
import functools

import jax
import jax.numpy as jnp
from jax import lax
from jax.experimental import pallas as pl
from jax.experimental.pallas import tpu as pltpu
from jax.experimental.pallas import tpu_sc as plsc

B = 16384
D = 64
NC = 2
NS = 16
NW = NC * NS
V = 1000000
VPAD = 1000064
WLANES = 256
TW = (V - 1) // WLANES
CS_MAX = VPAD - WLANES
P = TW // NW + 1
ORING = 16
SENT = 127 << 22

_sc_mesh = plsc.VectorSubcoreMesh(core_axis_name="c", subcore_axis_name="s")


@functools.partial(
    pl.kernel,
    out_type=[
        jax.ShapeDtypeStruct((B + 8, 128), jnp.float32),
        jax.ShapeDtypeStruct((B + 8, 128), jnp.float32),
    ],
    mesh=_sc_mesh,
    compiler_params=pltpu.CompilerParams(
        use_tc_tiling_on_sc=True, needs_layout_passes=False),
    scratch_types=[
        pltpu.VMEM((B,), jnp.int32),
        pltpu.VMEM((B + 16,), jnp.int32),
        pltpu.VMEM((B + 32,), jnp.int32),
        pltpu.VMEM((144,), jnp.int32),
        pltpu.VMEM((144,), jnp.int32),
        pltpu.VMEM((144,), jnp.int32),
        pltpu.VMEM((2, D, WLANES), jnp.float32),
        pltpu.VMEM((8, 16, 128), jnp.float32),
        pltpu.SemaphoreType.DMA,
        pltpu.SemaphoreType.DMA,
    ],
)
def _sc_gather(uidx, bidx, utabT, btabT, uout, bout,
               stage, lst, slst, hist, ioff, wkoff, wbuf, orow, wsem, osem):
    wid = lax.axis_index("s") * NC + lax.axis_index("c")
    iota16 = lax.iota(jnp.int32, 16)
    lane0 = iota16 == 0

    def scalar(vec, k):
        return lax.squeeze(lax.slice(vec, (k,), (k + 1,)), (0,))

    def splat_at(ref, i):
        return scalar(plsc.load_gather(ref, [jnp.full((16,), i, jnp.int32)]), 0)

    def route(idx_hbm):
        pltpu.sync_copy(idx_hbm, stage)

        def rbody(j4, cnt):
            parts = []
            for u in range(4):
                j = j4 * 4 + u
                iv = stage[pl.ds(j * 16, 16)]
                t = lax.shift_right_logical(iv, 8)
                mask = lax.bitwise_and(t, 31) == wid
                packed = lax.bitwise_or(
                    lax.shift_left(lax.shift_right_logical(t, 5), 22),
                    lax.bitwise_or(
                        lax.shift_left(lax.bitwise_and(iv, 255), 14),
                        iota16 + j * 16))
                nm = plsc.all_reduce_population_count(mask)
                parts.append((packed, mask, nm))
            for packed, mask, nm in parts:
                plsc.store_compressed(lst.at[pl.ds(cnt, 16)], packed, mask=mask)
                cnt = cnt + scalar(nm, 0)
            return cnt

        cnt = lax.fori_loop(0, B // 64, rbody, jnp.int32(0))

        for j in range(9):
            hist[pl.ds(j * 16, 16)] = jnp.zeros((16,), jnp.int32)

        nch = lax.div(cnt + jnp.int32(15), jnp.int32(16))

        def hbody(j, carry):
            pv = lst[pl.ds(j * 16, 16)]
            vi = (iota16 + j * 16) < cnt
            p_vec = jnp.where(vi, lax.shift_right_logical(pv, 22), 0)
            rank, lastm = plsc.scan_count(p_vec, mask=vi)
            cur = plsc.load_gather(hist, [p_vec])
            plsc.store_scatter(hist, [p_vec], cur + rank,
                               mask=jnp.logical_and(lastm, vi))
            return carry

        lax.fori_loop(0, nch, hbody, 0)

        def csum(j, carry):
            hv = hist[pl.ds(j * 16, 16)]
            cs = plsc.cumsum(hv) + carry
            ioff[pl.ds(j * 16, 16)] = cs
            wkoff[pl.ds(j * 16, 16)] = cs - hv
            return scalar(cs, 15)

        lax.fori_loop(0, 9, csum, jnp.int32(0))

        def sbody(j, carry):
            pv = lst[pl.ds(j * 16, 16)]
            vi = (iota16 + j * 16) < cnt
            p_vec = jnp.where(vi, lax.shift_right_logical(pv, 22), 0)
            rank, lastm = plsc.scan_count(p_vec, mask=vi)
            base = plsc.load_gather(wkoff, [p_vec])
            plsc.store_scatter(slst, [jnp.where(vi, base + rank - 1, 0)], pv,
                               mask=vi)
            plsc.store_scatter(wkoff, [p_vec], base + rank,
                               mask=jnp.logical_and(lastm, vi))
            return carry

        lax.fori_loop(0, nch, sbody, 0)
        slst[pl.ds(cnt, 16)] = jnp.full((16,), SENT, jnp.int32)
        return cnt

    def run(tabT, out_hbm, ocnt0):
        def fetch(p, slot):
            t = jnp.minimum(p * NW + wid, TW)
            cs = pl.multiple_of(jnp.minimum(t * WLANES, CS_MAX), 128)
            pltpu.async_copy(tabT.at[:, pl.ds(cs, WLANES)], wbuf.at[slot], wsem)

        fetch(0, 0)

        def pbody(p, ocnt):
            slot = lax.rem(p, 2)

            @pl.when(p + 1 < P)
            def _():
                fetch(p + 1, 1 - slot)

            pltpu.make_async_copy(
                tabT.at[:, pl.ds(0, WLANES)], wbuf.at[slot], wsem).wait()
            t = jnp.minimum(p * NW + wid, TW)
            cs = jnp.minimum(t * WLANES, CS_MAX)
            tcor = t * WLANES - cs
            seg_s = jnp.where(p > 0, splat_at(ioff, jnp.maximum(p - 1, 0)),
                              jnp.int32(0))
            seg_e = splat_at(ioff, p)

            def cbody(j, ocnt):
                pv = slst[pl.ds(j * 16, 16)]
                pmatch = lax.shift_right_logical(pv, 22) == p
                off_vec = lax.bitwise_and(
                    lax.shift_right_logical(pv, 14), 255) + tcor
                off_vec = jnp.where(pmatch, off_vec, 0)
                pos_vec = jnp.where(pmatch,
                                    lax.bitwise_and(pv, jnp.int32(16383)),
                                    jnp.int32(B))
                oslot = lax.rem(ocnt, 8)

                @pl.when(ocnt >= 8)
                def _():
                    pltpu.make_async_copy(
                        out_hbm.at[pl.ds(0, 16)], orow.at[oslot], osem).wait()

                for r in range(D):
                    vals = plsc.load_gather(
                        wbuf.at[slot],
                        [jnp.full((16,), r, jnp.int32), off_vec])
                    plsc.store_scatter(
                        orow.at[oslot],
                        [iota16, jnp.full((16,), r, jnp.int32)], vals)
                pltpu.async_copy(orow.at[oslot], out_hbm.at[pos_vec], osem)
                return ocnt + 1

            return lax.fori_loop(
                lax.shift_right_logical(seg_s, 4),
                lax.shift_right_logical(seg_e + jnp.int32(15), 4),
                cbody, ocnt)

        return lax.fori_loop(0, P, pbody, ocnt0)

    route(uidx)
    ocnt = run(utabT, uout, jnp.int32(0))
    route(bidx)
    ocnt = run(btabT, bout, ocnt)

    def dbody(m, carry):
        pltpu.make_async_copy(
            uout.at[pl.ds(0, 16)], orow.at[0], osem).wait()
        return carry

    lax.fori_loop(0, jnp.minimum(ocnt, 8), dbody, 0)


BLK = 2048


def _mlp_body(u_ref, b_ref, w1u_ref, w1b_ref, b1_ref, w2_ref, b2_ref,
              w3_ref, b3_ref, w4_ref, b4_ref, out_ref):
    u = u_ref[:, :D]
    b = b_ref[:, :D]
    h = (jnp.dot(u, w1u_ref[...], preferred_element_type=jnp.float32)
         + jnp.dot(b, w1b_ref[...], preferred_element_type=jnp.float32)
         + b1_ref[...])
    h = jnp.maximum(h, 0.0)
    h = jnp.dot(h, w2_ref[...], preferred_element_type=jnp.float32) + b2_ref[...]
    h = jnp.maximum(h, 0.0)
    h = jnp.dot(h, w3_ref[...], preferred_element_type=jnp.float32) + b3_ref[...]
    h = jnp.maximum(h, 0.0)
    o = jnp.dot(h, w4_ref[...], preferred_element_type=jnp.float32) + b4_ref[...]
    out_ref[...] = jax.nn.sigmoid(o)


_mlp_call = pl.pallas_call(
    _mlp_body,
    grid=(B // BLK,),
    in_specs=[
        pl.BlockSpec((BLK, 128), lambda i: (i, 0)),
        pl.BlockSpec((BLK, 128), lambda i: (i, 0)),
        pl.BlockSpec((D, 128), lambda i: (0, 0)),
        pl.BlockSpec((D, 128), lambda i: (0, 0)),
        pl.BlockSpec((1, 128), lambda i: (0, 0)),
        pl.BlockSpec((128, 64), lambda i: (0, 0)),
        pl.BlockSpec((1, 64), lambda i: (0, 0)),
        pl.BlockSpec((64, 32), lambda i: (0, 0)),
        pl.BlockSpec((1, 32), lambda i: (0, 0)),
        pl.BlockSpec((32, 1), lambda i: (0, 0)),
        pl.BlockSpec((1, 1), lambda i: (0, 0)),
    ],
    out_specs=pl.BlockSpec((BLK, 1), lambda i: (i, 0)),
    out_shape=jax.ShapeDtypeStruct((B, 1), jnp.float32),
)


def kernel(user, business, user_table, business_table, W1, b1, W2, b2, W3, b3, W4, b4):
    uidx = user.astype(jnp.int32)
    bidx = business.astype(jnp.int32)
    u_emb, b_emb = _sc_gather(uidx, bidx, user_table.T, business_table.T)
    w1ut = W1[:, :D].T
    w1bt = W1[:, D:].T
    out = _mlp_call(u_emb, b_emb, w1ut, w1bt, b1.reshape(1, 128),
                    W2.T, b2.reshape(1, 64), W3.T, b3.reshape(1, 32),
                    W4.T, b4.reshape(1, 1))
    return out[:, 0]


# speedup vs baseline: 16.2876x; 16.1870x over previous
import functools

import jax
import jax.numpy as jnp
from jax import lax
from jax.experimental import pallas as pl
from jax.experimental.pallas import tpu as pltpu
from jax.experimental.pallas import tpu_sc as plsc

B = 16384
D = 64
NC = 2
NS = 16
NW = NC * NS
V = 1000000
VPAD = 1000064
WLANES = 256
TW = (V - 1) // WLANES
CS_MAX = VPAD - WLANES
P = TW // NW + 1
ORING = 16
SENT = 127 << 22

_sc_mesh = plsc.VectorSubcoreMesh(core_axis_name="c", subcore_axis_name="s")


@functools.partial(
    pl.kernel,
    out_type=[
        jax.ShapeDtypeStruct((B, D), jnp.float32),
        jax.ShapeDtypeStruct((B, D), jnp.float32),
    ],
    mesh=_sc_mesh,
    compiler_params=pltpu.CompilerParams(
        use_tc_tiling_on_sc=True, needs_layout_passes=False),
    scratch_types=[
        pltpu.VMEM((B,), jnp.int32),
        pltpu.VMEM((B + 16,), jnp.int32),
        pltpu.VMEM((B + 32,), jnp.int32),
        pltpu.VMEM((144,), jnp.int32),
        pltpu.VMEM((144,), jnp.int32),
        pltpu.VMEM((144,), jnp.int32),
        pltpu.VMEM((2, D, WLANES), jnp.float32),
        pltpu.VMEM((ORING, D), jnp.float32),
        pltpu.SemaphoreType.DMA,
        pltpu.SemaphoreType.DMA,
    ],
)
def _sc_gather(uidx, bidx, utabT, btabT, uout, bout,
               stage, lst, slst, hist, ioff, wkoff, wbuf, orow, wsem, osem):
    wid = lax.axis_index("s") * NC + lax.axis_index("c")
    iota16 = lax.iota(jnp.int32, 16)
    lane0 = iota16 == 0

    def scalar(vec, k):
        return lax.squeeze(lax.slice(vec, (k,), (k + 1,)), (0,))

    def splat_at(ref, i):
        return scalar(plsc.load_gather(ref, [jnp.full((16,), i, jnp.int32)]), 0)

    def route(idx_hbm):
        pltpu.sync_copy(idx_hbm, stage)

        def rbody(j4, cnt):
            parts = []
            for u in range(4):
                j = j4 * 4 + u
                iv = stage[pl.ds(j * 16, 16)]
                t = lax.shift_right_logical(iv, 8)
                mask = lax.bitwise_and(t, 31) == wid
                packed = lax.bitwise_or(
                    lax.shift_left(lax.shift_right_logical(t, 5), 22),
                    lax.bitwise_or(
                        lax.shift_left(lax.bitwise_and(iv, 255), 14),
                        iota16 + j * 16))
                nm = plsc.all_reduce_population_count(mask)
                parts.append((packed, mask, nm))
            for packed, mask, nm in parts:
                plsc.store_compressed(lst.at[pl.ds(cnt, 16)], packed, mask=mask)
                cnt = cnt + scalar(nm, 0)
            return cnt

        cnt = lax.fori_loop(0, B // 64, rbody, jnp.int32(0))

        for j in range(9):
            hist[pl.ds(j * 16, 16)] = jnp.zeros((16,), jnp.int32)

        nch = lax.div(cnt + jnp.int32(15), jnp.int32(16))

        def hbody(j, carry):
            pv = lst[pl.ds(j * 16, 16)]
            vi = (iota16 + j * 16) < cnt
            p_vec = jnp.where(vi, lax.shift_right_logical(pv, 22), 0)
            rank, lastm = plsc.scan_count(p_vec, mask=vi)
            cur = plsc.load_gather(hist, [p_vec])
            plsc.store_scatter(hist, [p_vec], cur + rank,
                               mask=jnp.logical_and(lastm, vi))
            return carry

        lax.fori_loop(0, nch, hbody, 0)

        def csum(j, carry):
            hv = hist[pl.ds(j * 16, 16)]
            cs = plsc.cumsum(hv) + carry
            ioff[pl.ds(j * 16, 16)] = cs
            wkoff[pl.ds(j * 16, 16)] = cs - hv
            return scalar(cs, 15)

        lax.fori_loop(0, 9, csum, jnp.int32(0))

        def sbody(j, carry):
            pv = lst[pl.ds(j * 16, 16)]
            vi = (iota16 + j * 16) < cnt
            p_vec = jnp.where(vi, lax.shift_right_logical(pv, 22), 0)
            rank, lastm = plsc.scan_count(p_vec, mask=vi)
            base = plsc.load_gather(wkoff, [p_vec])
            plsc.store_scatter(slst, [jnp.where(vi, base + rank - 1, 0)], pv,
                               mask=vi)
            plsc.store_scatter(wkoff, [p_vec], base + rank,
                               mask=jnp.logical_and(lastm, vi))
            return carry

        lax.fori_loop(0, nch, sbody, 0)
        slst[pl.ds(cnt, 16)] = jnp.full((16,), SENT, jnp.int32)
        return cnt

    def run(tabT, out_hbm, ocnt0):
        def fetch(p, slot):
            t = jnp.minimum(p * NW + wid, TW)
            cs = pl.multiple_of(jnp.minimum(t * WLANES, CS_MAX), 128)
            pltpu.async_copy(tabT.at[:, pl.ds(cs, WLANES)], wbuf.at[slot], wsem)

        fetch(0, 0)

        def pbody(p, ocnt):
            slot = lax.rem(p, 2)

            @pl.when(p + 1 < P)
            def _():
                fetch(p + 1, 1 - slot)

            pltpu.make_async_copy(
                tabT.at[:, pl.ds(0, WLANES)], wbuf.at[slot], wsem).wait()
            t = jnp.minimum(p * NW + wid, TW)
            cs = jnp.minimum(t * WLANES, CS_MAX)
            tcor = t * WLANES - cs
            seg_s = jnp.where(p > 0, splat_at(ioff, jnp.maximum(p - 1, 0)),
                              jnp.int32(0))
            seg_e = splat_at(ioff, p)

            def cbody(j, ocnt):
                pv = slst[pl.ds(j * 16, 16)]
                pmi = (lax.shift_right_logical(pv, 22) == p).astype(jnp.int32)
                for k in range(16):
                    ck = scalar(pmi, k)
                    cond = ck > 0
                    oslot = lax.rem(ocnt, ORING)

                    @pl.when(jnp.logical_and(cond, ocnt >= ORING))
                    def _():
                        pltpu.make_async_copy(
                            out_hbm.at[0], orow.at[oslot], osem).wait()

                    @pl.when(cond)
                    def _():
                        v = scalar(pv, k)
                        off = lax.bitwise_and(
                            lax.shift_right_logical(v, 14), 255) + tcor
                        pos = lax.bitwise_and(v, jnp.int32(16383))
                        cols = jnp.full((16,), off, jnp.int32)
                        for q in range(D // 16):
                            rows = iota16 + q * 16
                            vals = plsc.load_gather(
                                wbuf.at[slot], [rows, cols])
                            orow[oslot, pl.ds(q * 16, 16)] = vals
                        pltpu.async_copy(
                            orow.at[oslot], out_hbm.at[pos], osem)

                    ocnt = ocnt + ck
                return ocnt

            return lax.fori_loop(
                lax.shift_right_logical(seg_s, 4),
                lax.shift_right_logical(seg_e + jnp.int32(15), 4),
                cbody, ocnt)

        return lax.fori_loop(0, P, pbody, ocnt0)

    route(uidx)
    ocnt = run(utabT, uout, jnp.int32(0))
    route(bidx)
    ocnt = run(btabT, bout, ocnt)

    def dbody(m, carry):
        pltpu.make_async_copy(uout.at[0], orow.at[0], osem).wait()
        return carry

    lax.fori_loop(0, jnp.minimum(ocnt, ORING), dbody, 0)


BLK = 2048


def _mlp_body(u_ref, b_ref, w1u_ref, w1b_ref, b1_ref, w2_ref, b2_ref,
              w3_ref, b3_ref, w4_ref, b4_ref, out_ref):
    u = u_ref[...]
    b = b_ref[...]
    h = (jnp.dot(u, w1u_ref[...], preferred_element_type=jnp.float32)
         + jnp.dot(b, w1b_ref[...], preferred_element_type=jnp.float32)
         + b1_ref[...])
    h = jnp.maximum(h, 0.0)
    h = jnp.dot(h, w2_ref[...], preferred_element_type=jnp.float32) + b2_ref[...]
    h = jnp.maximum(h, 0.0)
    h = jnp.dot(h, w3_ref[...], preferred_element_type=jnp.float32) + b3_ref[...]
    h = jnp.maximum(h, 0.0)
    o = jnp.dot(h, w4_ref[...], preferred_element_type=jnp.float32) + b4_ref[...]
    out_ref[...] = jax.nn.sigmoid(o)


_mlp_call = pl.pallas_call(
    _mlp_body,
    grid=(B // BLK,),
    in_specs=[
        pl.BlockSpec((BLK, D), lambda i: (i, 0)),
        pl.BlockSpec((BLK, D), lambda i: (i, 0)),
        pl.BlockSpec((D, 128), lambda i: (0, 0)),
        pl.BlockSpec((D, 128), lambda i: (0, 0)),
        pl.BlockSpec((1, 128), lambda i: (0, 0)),
        pl.BlockSpec((128, 64), lambda i: (0, 0)),
        pl.BlockSpec((1, 64), lambda i: (0, 0)),
        pl.BlockSpec((64, 32), lambda i: (0, 0)),
        pl.BlockSpec((1, 32), lambda i: (0, 0)),
        pl.BlockSpec((32, 1), lambda i: (0, 0)),
        pl.BlockSpec((1, 1), lambda i: (0, 0)),
    ],
    out_specs=pl.BlockSpec((BLK, 1), lambda i: (i, 0)),
    out_shape=jax.ShapeDtypeStruct((B, 1), jnp.float32),
)


def kernel(user, business, user_table, business_table, W1, b1, W2, b2, W3, b3, W4, b4):
    uidx = user.astype(jnp.int32)
    bidx = business.astype(jnp.int32)
    u_emb, b_emb = _sc_gather(uidx, bidx, user_table.T, business_table.T)
    w1ut = W1[:, :D].T
    w1bt = W1[:, D:].T
    out = _mlp_call(u_emb, b_emb, w1ut, w1bt, b1.reshape(1, 128),
                    W2.T, b2.reshape(1, 64), W3.T, b3.reshape(1, 32),
                    W4.T, b4.reshape(1, 1))
    return out[:, 0]


# speedup vs baseline: 20.0466x; 1.2308x over previous
import functools

import jax
import jax.numpy as jnp
from jax import lax
from jax.experimental import pallas as pl
from jax.experimental.pallas import tpu as pltpu
from jax.experimental.pallas import tpu_sc as plsc

B = 16384
D = 64
NC = 2
NS = 16
NW = NC * NS
V = 1000000
VPAD = 1000064
WLANES = 512
TW = (V - 1) // WLANES
CS_MAX = VPAD - WLANES
P = TW // NW + 1
ORING = 16
SENT = 127 << 23

_sc_mesh = plsc.VectorSubcoreMesh(core_axis_name="c", subcore_axis_name="s")


@functools.partial(
    pl.kernel,
    out_type=[
        jax.ShapeDtypeStruct((B, D), jnp.float32),
        jax.ShapeDtypeStruct((B, D), jnp.float32),
    ],
    mesh=_sc_mesh,
    compiler_params=pltpu.CompilerParams(
        use_tc_tiling_on_sc=True, needs_layout_passes=False),
    scratch_types=[
        pltpu.VMEM((B,), jnp.int32),
        pltpu.VMEM((B + 16,), jnp.int32),
        pltpu.VMEM((B + 32,), jnp.int32),
        pltpu.VMEM((144,), jnp.int32),
        pltpu.VMEM((144,), jnp.int32),
        pltpu.VMEM((144,), jnp.int32),
        pltpu.VMEM((2, D, WLANES), jnp.float32),
        pltpu.VMEM((ORING, D), jnp.float32),
        pltpu.SemaphoreType.DMA,
        pltpu.SemaphoreType.DMA,
    ],
)
def _sc_gather(uidx, bidx, utabT, btabT, uout, bout,
               stage, lst, slst, hist, ioff, wkoff, wbuf, orow, wsem, osem):
    wid = lax.axis_index("s") * NC + lax.axis_index("c")
    iota16 = lax.iota(jnp.int32, 16)
    lane0 = iota16 == 0

    def scalar(vec, k):
        return lax.squeeze(lax.slice(vec, (k,), (k + 1,)), (0,))

    def splat_at(ref, i):
        return scalar(plsc.load_gather(ref, [jnp.full((16,), i, jnp.int32)]), 0)

    def route(idx_hbm):
        pltpu.sync_copy(idx_hbm, stage)

        def rbody(j4, cnt):
            parts = []
            for u in range(4):
                j = j4 * 4 + u
                iv = stage[pl.ds(j * 16, 16)]
                t = lax.shift_right_logical(iv, 9)
                mask = lax.bitwise_and(t, 31) == wid
                packed = lax.bitwise_or(
                    lax.shift_left(lax.shift_right_logical(t, 5), 23),
                    lax.bitwise_or(
                        lax.shift_left(lax.bitwise_and(iv, 511), 14),
                        iota16 + j * 16))
                nm = plsc.all_reduce_population_count(mask)
                parts.append((packed, mask, nm))
            for packed, mask, nm in parts:
                plsc.store_compressed(lst.at[pl.ds(cnt, 16)], packed, mask=mask)
                cnt = cnt + scalar(nm, 0)
            return cnt

        cnt = lax.fori_loop(0, B // 64, rbody, jnp.int32(0))

        for j in range(9):
            hist[pl.ds(j * 16, 16)] = jnp.zeros((16,), jnp.int32)

        nch = lax.div(cnt + jnp.int32(15), jnp.int32(16))

        def hbody(j, carry):
            pv = lst[pl.ds(j * 16, 16)]
            vi = (iota16 + j * 16) < cnt
            p_vec = jnp.where(vi, lax.shift_right_logical(pv, 23), 0)
            rank, lastm = plsc.scan_count(p_vec, mask=vi)
            cur = plsc.load_gather(hist, [p_vec])
            plsc.store_scatter(hist, [p_vec], cur + rank,
                               mask=jnp.logical_and(lastm, vi))
            return carry

        lax.fori_loop(0, nch, hbody, 0)

        def csum(j, carry):
            hv = hist[pl.ds(j * 16, 16)]
            cs = plsc.cumsum(hv) + carry
            ioff[pl.ds(j * 16, 16)] = cs
            wkoff[pl.ds(j * 16, 16)] = cs - hv
            return scalar(cs, 15)

        lax.fori_loop(0, 9, csum, jnp.int32(0))

        def sbody(j, carry):
            pv = lst[pl.ds(j * 16, 16)]
            vi = (iota16 + j * 16) < cnt
            p_vec = jnp.where(vi, lax.shift_right_logical(pv, 23), 0)
            rank, lastm = plsc.scan_count(p_vec, mask=vi)
            base = plsc.load_gather(wkoff, [p_vec])
            plsc.store_scatter(slst, [jnp.where(vi, base + rank - 1, 0)], pv,
                               mask=vi)
            plsc.store_scatter(wkoff, [p_vec], base + rank,
                               mask=jnp.logical_and(lastm, vi))
            return carry

        lax.fori_loop(0, nch, sbody, 0)
        slst[pl.ds(cnt, 16)] = jnp.full((16,), SENT, jnp.int32)
        return cnt

    def run(tabT, out_hbm, ocnt0):
        def fetch(p, slot):
            t = jnp.minimum(p * NW + wid, TW)
            cs = pl.multiple_of(jnp.minimum(t * WLANES, CS_MAX), 128)
            pltpu.async_copy(tabT.at[:, pl.ds(cs, WLANES)], wbuf.at[slot], wsem)

        fetch(0, 0)

        def pbody(p, ocnt):
            slot = lax.rem(p, 2)

            @pl.when(p + 1 < P)
            def _():
                fetch(p + 1, 1 - slot)

            pltpu.make_async_copy(
                tabT.at[:, pl.ds(0, WLANES)], wbuf.at[slot], wsem).wait()
            t = jnp.minimum(p * NW + wid, TW)
            cs = jnp.minimum(t * WLANES, CS_MAX)
            tcor = t * WLANES - cs
            seg_s = jnp.where(p > 0, splat_at(ioff, jnp.maximum(p - 1, 0)),
                              jnp.int32(0))
            seg_e = splat_at(ioff, p)

            def cbody(j, ocnt):
                pv = slst[pl.ds(j * 16, 16)]
                pmi = (lax.shift_right_logical(pv, 23) == p).astype(jnp.int32)
                for k in range(16):
                    ck = scalar(pmi, k)
                    cond = ck > 0
                    oslot = lax.rem(ocnt, ORING)

                    @pl.when(jnp.logical_and(cond, ocnt >= ORING))
                    def _():
                        pltpu.make_async_copy(
                            out_hbm.at[0], orow.at[oslot], osem).wait()

                    @pl.when(cond)
                    def _():
                        v = scalar(pv, k)
                        off = lax.bitwise_and(
                            lax.shift_right_logical(v, 14), 511) + tcor
                        pos = lax.bitwise_and(v, jnp.int32(16383))
                        cols = jnp.full((16,), off, jnp.int32)
                        for q in range(D // 16):
                            rows = iota16 + q * 16
                            vals = plsc.load_gather(
                                wbuf.at[slot], [rows, cols])
                            orow[oslot, pl.ds(q * 16, 16)] = vals
                        pltpu.async_copy(
                            orow.at[oslot], out_hbm.at[pos], osem)

                    ocnt = ocnt + ck
                return ocnt

            return lax.fori_loop(
                lax.shift_right_logical(seg_s, 4),
                lax.shift_right_logical(seg_e + jnp.int32(15), 4),
                cbody, ocnt)

        return lax.fori_loop(0, P, pbody, ocnt0)

    route(uidx)
    ocnt = run(utabT, uout, jnp.int32(0))
    route(bidx)
    ocnt = run(btabT, bout, ocnt)

    def dbody(m, carry):
        pltpu.make_async_copy(uout.at[0], orow.at[0], osem).wait()
        return carry

    lax.fori_loop(0, jnp.minimum(ocnt, ORING), dbody, 0)


BLK = 2048


def _mlp_body(u_ref, b_ref, w1u_ref, w1b_ref, b1_ref, w2_ref, b2_ref,
              w3_ref, b3_ref, w4_ref, b4_ref, out_ref):
    u = u_ref[...]
    b = b_ref[...]
    h = (jnp.dot(u, w1u_ref[...], preferred_element_type=jnp.float32)
         + jnp.dot(b, w1b_ref[...], preferred_element_type=jnp.float32)
         + b1_ref[...])
    h = jnp.maximum(h, 0.0)
    h = jnp.dot(h, w2_ref[...], preferred_element_type=jnp.float32) + b2_ref[...]
    h = jnp.maximum(h, 0.0)
    h = jnp.dot(h, w3_ref[...], preferred_element_type=jnp.float32) + b3_ref[...]
    h = jnp.maximum(h, 0.0)
    o = jnp.dot(h, w4_ref[...], preferred_element_type=jnp.float32) + b4_ref[...]
    out_ref[...] = jax.nn.sigmoid(o)


_mlp_call = pl.pallas_call(
    _mlp_body,
    grid=(B // BLK,),
    in_specs=[
        pl.BlockSpec((BLK, D), lambda i: (i, 0)),
        pl.BlockSpec((BLK, D), lambda i: (i, 0)),
        pl.BlockSpec((D, 128), lambda i: (0, 0)),
        pl.BlockSpec((D, 128), lambda i: (0, 0)),
        pl.BlockSpec((1, 128), lambda i: (0, 0)),
        pl.BlockSpec((128, 64), lambda i: (0, 0)),
        pl.BlockSpec((1, 64), lambda i: (0, 0)),
        pl.BlockSpec((64, 32), lambda i: (0, 0)),
        pl.BlockSpec((1, 32), lambda i: (0, 0)),
        pl.BlockSpec((32, 1), lambda i: (0, 0)),
        pl.BlockSpec((1, 1), lambda i: (0, 0)),
    ],
    out_specs=pl.BlockSpec((BLK, 1), lambda i: (i, 0)),
    out_shape=jax.ShapeDtypeStruct((B, 1), jnp.float32),
)


def kernel(user, business, user_table, business_table, W1, b1, W2, b2, W3, b3, W4, b4):
    uidx = user.astype(jnp.int32)
    bidx = business.astype(jnp.int32)
    u_emb, b_emb = _sc_gather(uidx, bidx, user_table.T, business_table.T)
    w1ut = W1[:, :D].T
    w1bt = W1[:, D:].T
    out = _mlp_call(u_emb, b_emb, w1ut, w1bt, b1.reshape(1, 128),
                    W2.T, b2.reshape(1, 64), W3.T, b3.reshape(1, 32),
                    W4.T, b4.reshape(1, 1))
    return out[:, 0]
